# Initial kernel scaffold; baseline (speedup 1.0000x reference)
import functools

import jax
import jax.numpy as jnp
from jax import lax
from jax.experimental import pallas as pl
from jax.experimental.pallas import tpu as pltpu
from jax.experimental.pallas import tpu_sc as plsc

NUM_ENT = 100000
R = 16
EMB = 128
HEADS = 4
N = 10000
E = 320000
B = 1024
LIN1 = 128
MARGIN = 1.0

NC, NS, L = 2, 16, 16
NW = NC * NS

_MESH = plsc.VectorSubcoreMesh(core_axis_name="c", subcore_axis_name="s")

N_PAD = 10240
G_TOT = N_PAD + 2 * B
GPW = G_TOT // NW


def _gather_body(table, ids, out, idx_v, rows_v):
    wid = lax.axis_index("s") * NC + lax.axis_index("c")
    base = wid * GPW
    pltpu.sync_copy(ids.at[pl.ds(base, GPW)], idx_v)
    pltpu.sync_copy(table.at[idx_v], rows_v)
    pltpu.sync_copy(rows_v, out.at[pl.ds(base, GPW)])


_gather_rows = pl.kernel(
    _gather_body,
    out_type=jax.ShapeDtypeStruct((G_TOT, EMB), jnp.float32),
    mesh=_MESH,
    scratch_types=[
        pltpu.VMEM((GPW,), jnp.int32),
        pltpu.VMEM((GPW, EMB), jnp.float32),
    ],
)

EPC = E // NC
EPS = EPC // NS
ECH = 400
NCH = EPS // ECH
RPS = N // NS


def _counts_body(dst_h, et_h, out, oh_v, dst_v, et_v, cnt_sh):
    cid = lax.axis_index("c")
    sid = lax.axis_index("s")
    zero16 = jnp.zeros((L,), jnp.float32)
    ones16 = jnp.ones((L,), jnp.float32)
    iota16 = lax.iota(jnp.int32, L)

    def _zero_oh(i, _):
        oh_v[i, :] = zero16
        return 0

    lax.fori_loop(0, ECH, _zero_oh, 0)
    pltpu.sync_copy(oh_v, cnt_sh.at[pl.ds(sid * RPS, ECH)])
    pltpu.sync_copy(oh_v.at[pl.ds(0, RPS - ECH)],
                    cnt_sh.at[pl.ds(sid * RPS + ECH, RPS - ECH)])
    plsc.subcore_barrier()

    ebase = cid * EPC + sid * EPS

    def _chunk(ch, _):
        off = ebase + ch * ECH
        pltpu.sync_copy(dst_h.at[pl.ds(off, ECH)], dst_v)
        pltpu.sync_copy(et_h.at[pl.ds(off, ECH)], et_v)

        def _ones(j, _):
            rows = iota16 + j * L
            et = et_v[pl.ds(j * L, L)]
            plsc.store_scatter(oh_v, [rows, et], ones16)
            return 0

        lax.fori_loop(0, ECH // L, _ones, 0, unroll=4)
        pltpu.sync_copy(oh_v, cnt_sh.at[dst_v], add=True)

        def _zeros(j, _):
            rows = iota16 + j * L
            et = et_v[pl.ds(j * L, L)]
            plsc.store_scatter(oh_v, [rows, et], zero16)
            return 0

        lax.fori_loop(0, ECH // L, _zeros, 0, unroll=4)
        return 0

    lax.fori_loop(0, NCH, _chunk, 0)
    plsc.subcore_barrier()
    pltpu.sync_copy(cnt_sh.at[pl.ds(sid * RPS, RPS)],
                    out.at[cid, pl.ds(sid * RPS, RPS)])


_edge_counts = pl.kernel(
    _counts_body,
    out_type=jax.ShapeDtypeStruct((NC, N, R), jnp.float32),
    mesh=_MESH,
    scratch_types=[
        pltpu.VMEM((ECH, R), jnp.float32),
        pltpu.VMEM((ECH,), jnp.int32),
        pltpu.VMEM((ECH,), jnp.int32),
        pltpu.VMEM_SHARED((N, R), jnp.float32),
    ],
)


def _weights_body(c_ref, o_ref):
    c = c_ref[0] + c_ref[1]
    o_ref[...] = 1.0 / jnp.maximum(c, 1.0)


def _weights(counts):
    return pl.pallas_call(
        _weights_body,
        out_shape=jax.ShapeDtypeStruct((N, R), jnp.float32),
    )(counts)


NBLK = 10
BLK = N // NBLK


def _einsum_body(x_ref, w_ref, o_ref):
    x = x_ref[...]
    for r in range(R):
        o_ref[r] = jnp.dot(x, w_ref[r], preferred_element_type=jnp.float32)


def _einsum(x, conv_W):
    return pl.pallas_call(
        _einsum_body,
        grid=(NBLK,),
        in_specs=[
            pl.BlockSpec((BLK, EMB), lambda j: (j, 0)),
            pl.BlockSpec((R, EMB, EMB), lambda j: (0, 0, 0)),
        ],
        out_specs=pl.BlockSpec((R, BLK, EMB), lambda j: (0, j, 0)),
        out_shape=jax.ShapeDtypeStruct((R, N, EMB), jnp.float32),
    )(x, conv_W)


def _agg_body(tr_h, src_h, dst_h, et_h, w_h, out,
              gidx_v, src_v, dst_v, et_v, wrow_v, w_v, msg_v, agg_sh):
    cid = lax.axis_index("c")
    sid = lax.axis_index("s")
    zero16 = jnp.zeros((L,), jnp.float32)
    iota16 = lax.iota(jnp.int32, L)

    def _zero_msg(i, _):
        for k in range(EMB // L):
            msg_v[i, pl.ds(k * L, L)] = zero16
        return 0

    lax.fori_loop(0, ECH, _zero_msg, 0)
    pltpu.sync_copy(msg_v, agg_sh.at[pl.ds(sid * RPS, ECH)])
    pltpu.sync_copy(msg_v.at[pl.ds(0, RPS - ECH)],
                    agg_sh.at[pl.ds(sid * RPS + ECH, RPS - ECH)])
    plsc.subcore_barrier()

    ebase = cid * EPC + sid * EPS

    def _chunk(ch, _):
        off = ebase + ch * ECH
        pltpu.sync_copy(src_h.at[pl.ds(off, ECH)], src_v)
        pltpu.sync_copy(dst_h.at[pl.ds(off, ECH)], dst_v)
        pltpu.sync_copy(et_h.at[pl.ds(off, ECH)], et_v)

        def _gi(j, _):
            s = src_v[pl.ds(j * L, L)]
            t = et_v[pl.ds(j * L, L)]
            gidx_v[pl.ds(j * L, L)] = t * N + s
            return 0

        lax.fori_loop(0, ECH // L, _gi, 0, unroll=4)
        pltpu.sync_copy(tr_h.at[gidx_v], msg_v)
        pltpu.sync_copy(w_h.at[dst_v], wrow_v)

        def _wsel(j, _):
            rows = iota16 + j * L
            t = et_v[pl.ds(j * L, L)]
            w_v[pl.ds(j * L, L)] = plsc.load_gather(wrow_v, [rows, t])
            return 0

        lax.fori_loop(0, ECH // L, _wsel, 0, unroll=4)

        def _scale(e, _):
            w = w_v[e]
            for k in range(EMB // L):
                sl = pl.ds(k * L, L)
                msg_v[e, sl] = msg_v[e, sl] * w
            return 0

        lax.fori_loop(0, ECH, _scale, 0, unroll=2)
        pltpu.sync_copy(msg_v, agg_sh.at[dst_v], add=True)
        return 0

    lax.fori_loop(0, NCH, _chunk, 0)
    plsc.subcore_barrier()
    pltpu.sync_copy(agg_sh.at[pl.ds(sid * RPS, RPS)],
                    out.at[cid, pl.ds(sid * RPS, RPS)])


_aggregate = pl.kernel(
    _agg_body,
    out_type=jax.ShapeDtypeStruct((NC, N, EMB), jnp.float32),
    mesh=_MESH,
    scratch_types=[
        pltpu.VMEM((ECH,), jnp.int32),
        pltpu.VMEM((ECH,), jnp.int32),
        pltpu.VMEM((ECH,), jnp.int32),
        pltpu.VMEM((ECH,), jnp.int32),
        pltpu.VMEM((ECH, R), jnp.float32),
        pltpu.VMEM((ECH,), jnp.float32),
        pltpu.VMEM((ECH, EMB), jnp.float32),
        pltpu.VMEM_SHARED((N, EMB), jnp.float32),
    ],
)


def _dense_body(x_ref, a0_ref, a1_ref, bid_ref, root_ref, cb_ref,
                w1_ref, b1_ref, w2_ref, b2_ref, ans_ref, cor_ref,
                loss_ref, gold_ref, corr_ref, g_acc):
    j = pl.program_id(0)

    @pl.when(j == 0)
    def _():
        g_acc[...] = jnp.zeros_like(g_acc)

    h = a0_ref[...] + a1_ref[...] + jnp.dot(
        x_ref[...], root_ref[...], preferred_element_type=jnp.float32) + cb_ref[...]
    h = jnp.maximum(h, 0.0)
    h = jnp.maximum(jnp.dot(h, w1_ref[...], preferred_element_type=jnp.float32)
                    + b1_ref[...], 0.0)
    h = jnp.dot(h, w2_ref[...], preferred_element_type=jnp.float32) + b2_ref[...]
    bid = bid_ref[0, 0, :]
    oh = (bid[:, None] == lax.broadcasted_iota(jnp.int32, (BLK, B), 1)
          ).astype(jnp.float32)
    g_acc[...] += lax.dot_general(oh, h, (((0,), (0,)), ((), ())),
                                  preferred_element_type=jnp.float32)

    @pl.when(j == NBLK - 1)
    def _():
        g = g_acc[...]
        ans = ans_ref[...]
        cor = cor_ref[...]
        an = jnp.sqrt(jnp.sum(ans * ans, axis=1, keepdims=True))
        cn = jnp.sqrt(jnp.sum(cor * cor, axis=1, keepdims=True))
        best_g = None
        best_c = None
        for hd in range(HEADS):
            q = g[:, hd * EMB:(hd + 1) * EMB]
            qn = jnp.sqrt(jnp.sum(q * q, axis=1, keepdims=True))
            sg = jnp.sum(q * ans, axis=1, keepdims=True) / (qn * an)
            sc = jnp.sum(q * cor, axis=1, keepdims=True) / (qn * cn)
            best_g = sg if best_g is None else jnp.maximum(best_g, sg)
            best_c = sc if best_c is None else jnp.maximum(best_c, sc)
        gold_ref[...] = best_g
        corr_ref[...] = best_c
        loss_ref[...] = jnp.maximum(0.0, best_c - best_g + MARGIN)


def _dense(x, a0, a1, bid3, conv_root, conv_bias, lin1_W, lin1_b,
           lin2_W, lin2_b, ans_emb, cor_emb):
    def blk(shape):
        return pl.BlockSpec(shape, lambda j: (0,) * len(shape))

    nodes = pl.BlockSpec((BLK, EMB), lambda j: (j, 0))
    out1 = jax.ShapeDtypeStruct((B, 1), jnp.float32)
    return pl.pallas_call(
        _dense_body,
        grid=(NBLK,),
        in_specs=[
            nodes, nodes, nodes,
            pl.BlockSpec((1, 1, BLK), lambda j: (j, 0, 0)),
            blk((EMB, EMB)), blk((1, EMB)),
            blk((EMB, LIN1)), blk((1, LIN1)),
            blk((LIN1, HEADS * EMB)), blk((1, HEADS * EMB)),
            blk((B, EMB)), blk((B, EMB)),
        ],
        out_specs=[blk((B, 1))] * 3,
        out_shape=[out1, out1, out1],
        scratch_shapes=[pltpu.VMEM((B, HEADS * EMB), jnp.float32)],
    )(x, a0, a1, bid3, conv_root, conv_bias, lin1_W, lin1_b,
      lin2_W, lin2_b, ans_emb, cor_emb)


def kernel(node_ids, edge_index, edge_type, batch_id, answers, corrupted,
           entity_table, conv_W, conv_root, conv_bias,
           lin1_W, lin1_b, lin2_W, lin2_b):
    ids = jnp.concatenate([
        node_ids, jnp.zeros((N_PAD - N,), jnp.int32), answers, corrupted])
    rows = _gather_rows(entity_table, ids)
    x = rows[:N]
    ans_emb = rows[N_PAD:N_PAD + B]
    cor_emb = rows[N_PAD + B:N_PAD + 2 * B]

    src = edge_index[0]
    dst = edge_index[1]
    counts = _edge_counts(dst, edge_type)
    wtab = _weights(counts)
    tr = _einsum(x, conv_W)
    agg = _aggregate(tr.reshape(R * N, EMB), src, dst, edge_type, wtab)

    bid3 = batch_id.reshape(NBLK, 1, BLK)
    loss, gold, corr = _dense(
        x, agg[0], agg[1], bid3, conv_root, conv_bias.reshape(1, EMB),
        lin1_W, lin1_b.reshape(1, LIN1), lin2_W,
        lin2_b.reshape(1, HEADS * EMB), ans_emb, cor_emb)
    return loss.reshape(B), gold.reshape(B), corr.reshape(B)


# speedup vs baseline: 11.7921x; 11.7921x over previous
import jax
import jax.numpy as jnp
from jax import lax
from jax.experimental import pallas as pl
from jax.experimental.pallas import tpu as pltpu
from jax.experimental.pallas import tpu_sc as plsc

NUM_ENT = 100000
R = 16
EMB = 128
HEADS = 4
N = 10000
E = 320000
B = 1024
LIN1 = 128
MARGIN = 1.0

NC, NS, L = 2, 16, 16
NW = NC * NS

_MESH = plsc.VectorSubcoreMesh(core_axis_name="c", subcore_axis_name="s")
_SC_PARAMS = pltpu.CompilerParams(use_tc_tiling_on_sc=False,
                                  needs_layout_passes=False)

N_PAD = 10240
G_TOT = N_PAD + 2 * B
GPW = G_TOT // NW


def _gather_body(table, ids, out, idx_v, rows_v):
    wid = lax.axis_index("s") * NC + lax.axis_index("c")
    base = wid * GPW
    pltpu.sync_copy(ids.at[pl.ds(base, GPW)], idx_v)
    pltpu.sync_copy(table.at[idx_v], rows_v)
    pltpu.sync_copy(rows_v, out.at[pl.ds(base, GPW)])


_gather_rows = pl.kernel(
    _gather_body,
    out_type=jax.ShapeDtypeStruct((G_TOT, EMB), jnp.float32),
    mesh=_MESH,
    compiler_params=_SC_PARAMS,
    scratch_types=[
        pltpu.VMEM((GPW,), jnp.int32),
        pltpu.VMEM((GPW, EMB), jnp.float32),
    ],
)

EPC = E // NC
EPS = EPC // NS
ECH = 400
NCH = EPS // ECH
RPS = N // NS


def _counts_body(dst_h, et_h, c0, c1, oh_v, dst_v, et_v, cnt_sh):
    cid = lax.axis_index("c")
    sid = lax.axis_index("s")
    zero16 = jnp.zeros((L,), jnp.float32)
    ones16 = jnp.ones((L,), jnp.float32)
    iota16 = lax.iota(jnp.int32, L)

    def _zero_oh(i, _):
        oh_v[i, :] = zero16
        return 0

    lax.fori_loop(0, ECH, _zero_oh, 0)
    pltpu.sync_copy(oh_v, cnt_sh.at[pl.ds(sid * RPS, ECH)])
    pltpu.sync_copy(oh_v.at[pl.ds(0, RPS - ECH)],
                    cnt_sh.at[pl.ds(sid * RPS + ECH, RPS - ECH)])
    plsc.subcore_barrier()

    ebase = cid * EPC + sid * EPS

    def _chunk(ch, _):
        off = ebase + ch * ECH
        pltpu.sync_copy(dst_h.at[pl.ds(off, ECH)], dst_v)
        pltpu.sync_copy(et_h.at[pl.ds(off, ECH)], et_v)

        def _ones(j, _):
            rows = iota16 + j * L
            et = et_v[pl.ds(j * L, L)]
            plsc.store_scatter(oh_v, [rows, et], ones16)
            return 0

        lax.fori_loop(0, ECH // L, _ones, 0, unroll=4)
        pltpu.sync_copy(oh_v, cnt_sh.at[dst_v], add=True)

        def _zeros(j, _):
            rows = iota16 + j * L
            et = et_v[pl.ds(j * L, L)]
            plsc.store_scatter(oh_v, [rows, et], zero16)
            return 0

        lax.fori_loop(0, ECH // L, _zeros, 0, unroll=4)
        return 0

    lax.fori_loop(0, NCH, _chunk, 0)
    plsc.subcore_barrier()

    @pl.when(cid == 0)
    def _():
        pltpu.sync_copy(cnt_sh.at[pl.ds(sid * RPS, RPS)],
                        c0.at[pl.ds(sid * RPS, RPS)])

    @pl.when(cid == 1)
    def _():
        pltpu.sync_copy(cnt_sh.at[pl.ds(sid * RPS, RPS)],
                        c1.at[pl.ds(sid * RPS, RPS)])


_edge_counts = pl.kernel(
    _counts_body,
    out_type=[jax.ShapeDtypeStruct((N, R), jnp.float32)] * 2,
    mesh=_MESH,
    compiler_params=_SC_PARAMS,
    scratch_types=[
        pltpu.VMEM((ECH, R), jnp.float32),
        pltpu.VMEM((ECH,), jnp.int32),
        pltpu.VMEM((ECH,), jnp.int32),
        pltpu.VMEM_SHARED((N, R), jnp.float32),
    ],
)


NBLK = 10
BLK = N // NBLK


def _einsum_body(x_ref, w_ref, o_ref):
    x = x_ref[...]
    for r in range(R):
        o_ref[r] = jnp.dot(x, w_ref[r], preferred_element_type=jnp.float32)


def _einsum(x, conv_W):
    return pl.pallas_call(
        _einsum_body,
        grid=(NBLK,),
        in_specs=[
            pl.BlockSpec((BLK, EMB), lambda j: (j, 0)),
            pl.BlockSpec((R, EMB, EMB), lambda j: (0, 0, 0)),
        ],
        out_specs=pl.BlockSpec((R, BLK, EMB), lambda j: (0, j, 0)),
        out_shape=jax.ShapeDtypeStruct((R, N, EMB), jnp.float32),
    )(x, conv_W)


ACH = 80
ANCH = EPS // ACH


def _agg_body(tr_h, src_h, dst_h, et_h, c0_h, c1_h, a0, a1,
              gidx_v, src_v, dst_v, et_v, c0r_v, c1r_v, w_v, msg_v, agg_sh):
    cid = lax.axis_index("c")
    sid = lax.axis_index("s")
    zero16 = jnp.zeros((L,), jnp.float32)
    iota16 = lax.iota(jnp.int32, L)

    def _zero_msg(i, _):
        for k in range(EMB // L):
            msg_v[i, pl.ds(k * L, L)] = zero16
        return 0

    lax.fori_loop(0, ACH, _zero_msg, 0)
    for t in range(RPS // ACH):
        pltpu.sync_copy(msg_v, agg_sh.at[pl.ds(sid * RPS + t * ACH, ACH)])
    rem = RPS % ACH
    pltpu.sync_copy(msg_v.at[pl.ds(0, rem)],
                    agg_sh.at[pl.ds(sid * RPS + RPS - rem, rem)])
    plsc.subcore_barrier()

    ebase = cid * EPC + sid * EPS

    def _chunk(ch, _):
        off = ebase + ch * ACH
        pltpu.sync_copy(src_h.at[pl.ds(off, ACH)], src_v)
        pltpu.sync_copy(dst_h.at[pl.ds(off, ACH)], dst_v)
        pltpu.sync_copy(et_h.at[pl.ds(off, ACH)], et_v)

        def _gi(j, _):
            s = src_v[pl.ds(j * L, L)]
            t = et_v[pl.ds(j * L, L)]
            gidx_v[pl.ds(j * L, L)] = t * N + s
            return 0

        lax.fori_loop(0, ACH // L, _gi, 0, unroll=5)
        pltpu.sync_copy(tr_h.at[gidx_v], msg_v)
        pltpu.sync_copy(c0_h.at[dst_v], c0r_v)
        pltpu.sync_copy(c1_h.at[dst_v], c1r_v)

        def _wsel(j, _):
            rows = iota16 + j * L
            t = et_v[pl.ds(j * L, L)]
            c = (plsc.load_gather(c0r_v, [rows, t])
                 + plsc.load_gather(c1r_v, [rows, t]))
            w_v[pl.ds(j * L, L)] = 1.0 / jnp.maximum(c, 1.0)
            return 0

        lax.fori_loop(0, ACH // L, _wsel, 0, unroll=5)

        def _scale(jg, _):
            wv = w_v[pl.ds(jg * L, L)]
            for l in range(L):
                e = jg * L + l
                w = wv[l]
                for k in range(EMB // L):
                    sl = pl.ds(k * L, L)
                    msg_v[e, sl] = msg_v[e, sl] * w
            return 0

        lax.fori_loop(0, ACH // L, _scale, 0)
        pltpu.sync_copy(msg_v, agg_sh.at[dst_v], add=True)
        return 0

    lax.fori_loop(0, ANCH, _chunk, 0)
    plsc.subcore_barrier()

    @pl.when(cid == 0)
    def _():
        pltpu.sync_copy(agg_sh.at[pl.ds(sid * RPS, RPS)],
                        a0.at[pl.ds(sid * RPS, RPS)])

    @pl.when(cid == 1)
    def _():
        pltpu.sync_copy(agg_sh.at[pl.ds(sid * RPS, RPS)],
                        a1.at[pl.ds(sid * RPS, RPS)])


_aggregate = pl.kernel(
    _agg_body,
    out_type=[jax.ShapeDtypeStruct((N, EMB), jnp.float32)] * 2,
    mesh=_MESH,
    compiler_params=_SC_PARAMS,
    scratch_types=[
        pltpu.VMEM((ACH,), jnp.int32),
        pltpu.VMEM((ACH,), jnp.int32),
        pltpu.VMEM((ACH,), jnp.int32),
        pltpu.VMEM((ACH,), jnp.int32),
        pltpu.VMEM((ACH, R), jnp.float32),
        pltpu.VMEM((ACH, R), jnp.float32),
        pltpu.VMEM((ACH,), jnp.float32),
        pltpu.VMEM((ACH, EMB), jnp.float32),
        pltpu.VMEM_SHARED((N, EMB), jnp.float32),
    ],
)


def _dense_body(x_ref, a0_ref, a1_ref, bid_ref, root_ref, cb_ref,
                w1_ref, b1_ref, w2_ref, b2_ref, ans_ref, cor_ref,
                loss_ref, gold_ref, corr_ref, g_acc):
    j = pl.program_id(0)

    @pl.when(j == 0)
    def _():
        g_acc[...] = jnp.zeros_like(g_acc)

    h = a0_ref[...] + a1_ref[...] + jnp.dot(
        x_ref[...], root_ref[...], preferred_element_type=jnp.float32) + cb_ref[...]
    h = jnp.maximum(h, 0.0)
    h = jnp.maximum(jnp.dot(h, w1_ref[...], preferred_element_type=jnp.float32)
                    + b1_ref[...], 0.0)
    h = jnp.dot(h, w2_ref[...], preferred_element_type=jnp.float32) + b2_ref[...]
    bid = bid_ref[0, 0, :]
    oh = (bid[:, None] == lax.broadcasted_iota(jnp.int32, (BLK, B), 1)
          ).astype(jnp.float32)
    g_acc[...] += lax.dot_general(oh, h, (((0,), (0,)), ((), ())),
                                  preferred_element_type=jnp.float32)

    @pl.when(j == NBLK - 1)
    def _():
        g = g_acc[...]
        ans = ans_ref[...]
        cor = cor_ref[...]
        an = jnp.sqrt(jnp.sum(ans * ans, axis=1, keepdims=True))
        cn = jnp.sqrt(jnp.sum(cor * cor, axis=1, keepdims=True))
        best_g = None
        best_c = None
        for hd in range(HEADS):
            q = g[:, hd * EMB:(hd + 1) * EMB]
            qn = jnp.sqrt(jnp.sum(q * q, axis=1, keepdims=True))
            sg = jnp.sum(q * ans, axis=1, keepdims=True) / (qn * an)
            sc = jnp.sum(q * cor, axis=1, keepdims=True) / (qn * cn)
            best_g = sg if best_g is None else jnp.maximum(best_g, sg)
            best_c = sc if best_c is None else jnp.maximum(best_c, sc)
        gold_ref[...] = best_g
        corr_ref[...] = best_c
        loss_ref[...] = jnp.maximum(0.0, best_c - best_g + MARGIN)


def _dense(x, a0, a1, bid3, conv_root, conv_bias, lin1_W, lin1_b,
           lin2_W, lin2_b, ans_emb, cor_emb):
    def blk(shape):
        return pl.BlockSpec(shape, lambda j: (0,) * len(shape))

    nodes = pl.BlockSpec((BLK, EMB), lambda j: (j, 0))
    out1 = jax.ShapeDtypeStruct((B, 1), jnp.float32)
    return pl.pallas_call(
        _dense_body,
        grid=(NBLK,),
        in_specs=[
            nodes, nodes, nodes,
            pl.BlockSpec((1, 1, BLK), lambda j: (j, 0, 0)),
            blk((EMB, EMB)), blk((1, EMB)),
            blk((EMB, LIN1)), blk((1, LIN1)),
            blk((LIN1, HEADS * EMB)), blk((1, HEADS * EMB)),
            blk((B, EMB)), blk((B, EMB)),
        ],
        out_specs=[blk((B, 1))] * 3,
        out_shape=[out1, out1, out1],
        scratch_shapes=[pltpu.VMEM((B, HEADS * EMB), jnp.float32)],
    )(x, a0, a1, bid3, conv_root, conv_bias, lin1_W, lin1_b,
      lin2_W, lin2_b, ans_emb, cor_emb)


def kernel(node_ids, edge_index, edge_type, batch_id, answers, corrupted,
           entity_table, conv_W, conv_root, conv_bias,
           lin1_W, lin1_b, lin2_W, lin2_b):
    ids = jnp.concatenate([
        node_ids, jnp.zeros((N_PAD - N,), jnp.int32), answers, corrupted])
    rows = _gather_rows(entity_table, ids)
    x = rows[:N]
    ans_emb = rows[N_PAD:N_PAD + B]
    cor_emb = rows[N_PAD + B:N_PAD + 2 * B]

    src = edge_index[0]
    dst = edge_index[1]
    c0, c1 = _edge_counts(dst, edge_type)
    tr = _einsum(x, conv_W)
    a0, a1 = _aggregate(tr.reshape(R * N, EMB), src, dst, edge_type, c0, c1)

    bid3 = batch_id.reshape(NBLK, 1, BLK)
    loss, gold, corr = _dense(
        x, a0, a1, bid3, conv_root, conv_bias.reshape(1, EMB),
        lin1_W, lin1_b.reshape(1, LIN1), lin2_W,
        lin2_b.reshape(1, HEADS * EMB), ans_emb, cor_emb)
    return loss.reshape(B), gold.reshape(B), corr.reshape(B)


# speedup vs baseline: 25.4544x; 2.1586x over previous
import jax
import jax.numpy as jnp
from jax import lax
from jax.experimental import pallas as pl
from jax.experimental.pallas import tpu as pltpu
from jax.experimental.pallas import tpu_sc as plsc

NUM_ENT = 100000
R = 16
EMB = 128
HEADS = 4
N = 10000
E = 320000
B = 1024
LIN1 = 128
MARGIN = 1.0

NC, NS, L = 2, 16, 16
NW = NC * NS

_MESH = plsc.VectorSubcoreMesh(core_axis_name="c", subcore_axis_name="s")
_SC_PARAMS = pltpu.CompilerParams(use_tc_tiling_on_sc=False,
                                  needs_layout_passes=False)

N_PAD = 10240
G_TOT = N_PAD + 2 * B
GPW = G_TOT // NW


def _gather_body(table, ids, out, idx_v, rows_v):
    wid = lax.axis_index("s") * NC + lax.axis_index("c")
    base = wid * GPW
    pltpu.sync_copy(ids.at[pl.ds(base, GPW)], idx_v)
    pltpu.sync_copy(table.at[idx_v], rows_v)
    pltpu.sync_copy(rows_v, out.at[pl.ds(base, GPW)])


_gather_rows = pl.kernel(
    _gather_body,
    out_type=jax.ShapeDtypeStruct((G_TOT, EMB), jnp.float32),
    mesh=_MESH,
    compiler_params=_SC_PARAMS,
    scratch_types=[
        pltpu.VMEM((GPW,), jnp.int32),
        pltpu.VMEM((GPW, EMB), jnp.float32),
    ],
)

EPC = E // NC
EPS = EPC // NS
ECH = 400
NCH = EPS // ECH
RPS = N // NS


def _counts_body(dst_h, et_h, cboth, oh_v, dst_v, et_v, cnt_sh):
    cid = lax.axis_index("c")
    sid = lax.axis_index("s")
    zero16 = jnp.zeros((L,), jnp.float32)
    ones16 = jnp.ones((L,), jnp.float32)
    iota16 = lax.iota(jnp.int32, L)

    def _zero_oh(i, _):
        oh_v[i, :] = zero16
        return 0

    lax.fori_loop(0, ECH, _zero_oh, 0)
    pltpu.sync_copy(oh_v, cnt_sh.at[pl.ds(sid * RPS, ECH)])
    pltpu.sync_copy(oh_v.at[pl.ds(0, RPS - ECH)],
                    cnt_sh.at[pl.ds(sid * RPS + ECH, RPS - ECH)])
    plsc.subcore_barrier()

    ebase = cid * EPC + sid * EPS

    def _chunk(ch, _):
        off = ebase + ch * ECH
        pltpu.sync_copy(dst_h.at[pl.ds(off, ECH)], dst_v)
        pltpu.sync_copy(et_h.at[pl.ds(off, ECH)], et_v)

        def _ones(j, _):
            rows = iota16 + j * L
            et = et_v[pl.ds(j * L, L)]
            plsc.store_scatter(oh_v, [rows, et], ones16)
            return 0

        lax.fori_loop(0, ECH // L, _ones, 0, unroll=4)
        pltpu.sync_copy(oh_v, cnt_sh.at[dst_v], add=True)

        def _zeros(j, _):
            rows = iota16 + j * L
            et = et_v[pl.ds(j * L, L)]
            plsc.store_scatter(oh_v, [rows, et], zero16)
            return 0

        lax.fori_loop(0, ECH // L, _zeros, 0, unroll=4)
        return 0

    lax.fori_loop(0, NCH, _chunk, 0)
    plsc.subcore_barrier()
    pltpu.sync_copy(cnt_sh.at[pl.ds(sid * RPS, RPS)],
                    cboth.at[pl.ds(sid * RPS, RPS), pl.ds(cid * R, R)])


_edge_counts = pl.kernel(
    _counts_body,
    out_type=jax.ShapeDtypeStruct((N, NC * R), jnp.float32),
    mesh=_MESH,
    compiler_params=_SC_PARAMS,
    scratch_types=[
        pltpu.VMEM((ECH, R), jnp.float32),
        pltpu.VMEM((ECH,), jnp.int32),
        pltpu.VMEM((ECH,), jnp.int32),
        pltpu.VMEM_SHARED((N, R), jnp.float32),
    ],
)


NBLK = 10
BLK = N // NBLK


def _einsum_body(x_ref, w_ref, o_ref):
    x = x_ref[...]
    for r in range(R):
        o_ref[r] = jnp.dot(x, w_ref[r], preferred_element_type=jnp.float32)


def _einsum(x, conv_W):
    return pl.pallas_call(
        _einsum_body,
        grid=(NBLK,),
        in_specs=[
            pl.BlockSpec((BLK, EMB), lambda j: (j, 0)),
            pl.BlockSpec((R, EMB, EMB), lambda j: (0, 0, 0)),
        ],
        out_specs=pl.BlockSpec((R, BLK, EMB), lambda j: (0, j, 0)),
        out_shape=jax.ShapeDtypeStruct((R, N, EMB), jnp.float32),
    )(x, conv_W)


ACH = 80
ANCH = EPS // ACH


def _agg_body(tr_h, pk_h, cnt_h, a0, a1,
              pk0, pk1, gidx0, gidx1, dst0, dst1, crow0, crow1, w_v,
              msg0, msg1, agg_sh, spk0, spk1, sg0, sg1):
    cid = lax.axis_index("c")
    sid = lax.axis_index("s")
    zero16 = jnp.zeros((L,), jnp.float32)
    iota16 = lax.iota(jnp.int32, L)
    pk = (pk0, pk1)
    gidx = (gidx0, gidx1)
    dstb = (dst0, dst1)
    crow = (crow0, crow1)
    msg = (msg0, msg1)
    spk = (spk0, spk1)
    sg = (sg0, sg1)

    def _zero_msg(i, _):
        for k in range(EMB // L):
            msg0[i, pl.ds(k * L, L)] = zero16
        return 0

    lax.fori_loop(0, ACH, _zero_msg, 0)
    for t in range(RPS // ACH):
        pltpu.sync_copy(msg0, agg_sh.at[pl.ds(sid * RPS + t * ACH, ACH)])
    rem = RPS % ACH
    pltpu.sync_copy(msg0.at[pl.ds(0, rem)],
                    agg_sh.at[pl.ds(sid * RPS + RPS - rem, rem)])
    plsc.subcore_barrier()

    wkr = cid * NS + sid
    bbase = wkr * ANCH

    def _issue_pack(ch, p):
        pltpu.async_copy(pk_h.at[bbase + ch], pk[p], spk[p])

    def _wait_pack(p):
        pltpu.make_async_copy(pk_h.at[0], pk[p], spk[p]).wait()

    def _prep(ch, p):
        _wait_pack(p)

        def _gi(j, _):
            s = pk[p][0, pl.ds(j * L, L)]
            d = pk[p][1, pl.ds(j * L, L)]
            t = pk[p][2, pl.ds(j * L, L)]
            gidx[p][pl.ds(j * L, L)] = t * N + s
            dstb[p][pl.ds(j * L, L)] = d
            return 0

        lax.fori_loop(0, ACH // L, _gi, 0, unroll=5)
        pltpu.async_copy(tr_h.at[gidx[p]], msg[p], sg[p])
        pltpu.async_copy(cnt_h.at[dstb[p]], crow[p], sg[p])

    def _process(p):
        pltpu.make_async_copy(tr_h.at[pl.ds(0, ACH)], msg[p], sg[p]).wait()
        pltpu.make_async_copy(cnt_h.at[pl.ds(0, ACH)], crow[p], sg[p]).wait()

        def _wsel(j, _):
            rows = iota16 + j * L
            t = pk[p][2, pl.ds(j * L, L)]
            c = (plsc.load_gather(crow[p], [rows, t])
                 + plsc.load_gather(crow[p], [rows, t + R]))
            w_v[pl.ds(j * L, L)] = 1.0 / jnp.maximum(c, 1.0)
            return 0

        lax.fori_loop(0, ACH // L, _wsel, 0, unroll=5)

        def _scale(jg, _):
            wv = w_v[pl.ds(jg * L, L)]
            for l in range(L):
                e = jg * L + l
                w = wv[l]
                for k in range(EMB // L):
                    sl = pl.ds(k * L, L)
                    msg[p][e, sl] = msg[p][e, sl] * w
            return 0

        lax.fori_loop(0, ACH // L, _scale, 0)
        pltpu.sync_copy(msg[p], agg_sh.at[dstb[p]], add=True)

    _issue_pack(0, 0)
    _issue_pack(1, 1)
    _prep(0, 0)

    def _pair(it, _):
        ch = it * 2
        for p in range(2):
            c = ch + p
            q = 1 - p

            @pl.when(c + 1 < ANCH)
            def _():
                _prep(c + 1, q)

            @pl.when(c < ANCH)
            def _():
                _process(p)

            @pl.when(c + 2 < ANCH)
            def _():
                _issue_pack(c + 2, p)

        return 0

    lax.fori_loop(0, (ANCH + 1) // 2, _pair, 0)
    plsc.subcore_barrier()

    @pl.when(cid == 0)
    def _():
        pltpu.sync_copy(agg_sh.at[pl.ds(sid * RPS, RPS)],
                        a0.at[pl.ds(sid * RPS, RPS)])

    @pl.when(cid == 1)
    def _():
        pltpu.sync_copy(agg_sh.at[pl.ds(sid * RPS, RPS)],
                        a1.at[pl.ds(sid * RPS, RPS)])


_aggregate = pl.kernel(
    _agg_body,
    out_type=[jax.ShapeDtypeStruct((N, EMB), jnp.float32)] * 2,
    mesh=_MESH,
    compiler_params=_SC_PARAMS,
    scratch_types=[
        pltpu.VMEM((3, ACH), jnp.int32),
        pltpu.VMEM((3, ACH), jnp.int32),
        pltpu.VMEM((ACH,), jnp.int32),
        pltpu.VMEM((ACH,), jnp.int32),
        pltpu.VMEM((ACH,), jnp.int32),
        pltpu.VMEM((ACH,), jnp.int32),
        pltpu.VMEM((ACH, NC * R), jnp.float32),
        pltpu.VMEM((ACH, NC * R), jnp.float32),
        pltpu.VMEM((ACH,), jnp.float32),
        pltpu.VMEM((ACH, EMB), jnp.float32),
        pltpu.VMEM((ACH, EMB), jnp.float32),
        pltpu.VMEM_SHARED((N, EMB), jnp.float32),
        pltpu.SemaphoreType.DMA,
        pltpu.SemaphoreType.DMA,
        pltpu.SemaphoreType.DMA,
        pltpu.SemaphoreType.DMA,
    ],
)


def _dense_body(x_ref, a0_ref, a1_ref, bid_ref, root_ref, cb_ref,
                w1_ref, b1_ref, w2_ref, b2_ref, ans_ref, cor_ref,
                loss_ref, gold_ref, corr_ref, g_acc):
    j = pl.program_id(0)

    @pl.when(j == 0)
    def _():
        g_acc[...] = jnp.zeros_like(g_acc)

    h = a0_ref[...] + a1_ref[...] + jnp.dot(
        x_ref[...], root_ref[...], preferred_element_type=jnp.float32) + cb_ref[...]
    h = jnp.maximum(h, 0.0)
    h = jnp.maximum(jnp.dot(h, w1_ref[...], preferred_element_type=jnp.float32)
                    + b1_ref[...], 0.0)
    h = jnp.dot(h, w2_ref[...], preferred_element_type=jnp.float32) + b2_ref[...]
    bid = bid_ref[0, 0, :]
    oh = (bid[:, None] == lax.broadcasted_iota(jnp.int32, (BLK, B), 1)
          ).astype(jnp.float32)
    g_acc[...] += lax.dot_general(oh, h, (((0,), (0,)), ((), ())),
                                  preferred_element_type=jnp.float32)

    @pl.when(j == NBLK - 1)
    def _():
        g = g_acc[...]
        ans = ans_ref[...]
        cor = cor_ref[...]
        an = jnp.sqrt(jnp.sum(ans * ans, axis=1, keepdims=True))
        cn = jnp.sqrt(jnp.sum(cor * cor, axis=1, keepdims=True))
        best_g = None
        best_c = None
        for hd in range(HEADS):
            q = g[:, hd * EMB:(hd + 1) * EMB]
            qn = jnp.sqrt(jnp.sum(q * q, axis=1, keepdims=True))
            sg = jnp.sum(q * ans, axis=1, keepdims=True) / (qn * an)
            sc = jnp.sum(q * cor, axis=1, keepdims=True) / (qn * cn)
            best_g = sg if best_g is None else jnp.maximum(best_g, sg)
            best_c = sc if best_c is None else jnp.maximum(best_c, sc)
        gold_ref[...] = best_g
        corr_ref[...] = best_c
        loss_ref[...] = jnp.maximum(0.0, best_c - best_g + MARGIN)


def _dense(x, a0, a1, bid3, conv_root, conv_bias, lin1_W, lin1_b,
           lin2_W, lin2_b, ans_emb, cor_emb):
    def blk(shape):
        return pl.BlockSpec(shape, lambda j: (0,) * len(shape))

    nodes = pl.BlockSpec((BLK, EMB), lambda j: (j, 0))
    out1 = jax.ShapeDtypeStruct((B, 1), jnp.float32)
    return pl.pallas_call(
        _dense_body,
        grid=(NBLK,),
        in_specs=[
            nodes, nodes, nodes,
            pl.BlockSpec((1, 1, BLK), lambda j: (j, 0, 0)),
            blk((EMB, EMB)), blk((1, EMB)),
            blk((EMB, LIN1)), blk((1, LIN1)),
            blk((LIN1, HEADS * EMB)), blk((1, HEADS * EMB)),
            blk((B, EMB)), blk((B, EMB)),
        ],
        out_specs=[blk((B, 1))] * 3,
        out_shape=[out1, out1, out1],
        scratch_shapes=[pltpu.VMEM((B, HEADS * EMB), jnp.float32)],
    )(x, a0, a1, bid3, conv_root, conv_bias, lin1_W, lin1_b,
      lin2_W, lin2_b, ans_emb, cor_emb)


def kernel(node_ids, edge_index, edge_type, batch_id, answers, corrupted,
           entity_table, conv_W, conv_root, conv_bias,
           lin1_W, lin1_b, lin2_W, lin2_b):
    ids = jnp.concatenate([
        node_ids, jnp.zeros((N_PAD - N,), jnp.int32), answers, corrupted])
    rows = _gather_rows(entity_table, ids)
    x = rows[:N]
    ans_emb = rows[N_PAD:N_PAD + B]
    cor_emb = rows[N_PAD + B:N_PAD + 2 * B]

    src = edge_index[0]
    dst = edge_index[1]
    counts = _edge_counts(dst, edge_type)
    tr = _einsum(x, conv_W)
    packed = (jnp.stack([src, dst, edge_type])
              .reshape(3, E // ACH, ACH).transpose(1, 0, 2))
    a0, a1 = _aggregate(tr.reshape(R * N, EMB), packed, counts)

    bid3 = batch_id.reshape(NBLK, 1, BLK)
    loss, gold, corr = _dense(
        x, a0, a1, bid3, conv_root, conv_bias.reshape(1, EMB),
        lin1_W, lin1_b.reshape(1, LIN1), lin2_W,
        lin2_b.reshape(1, HEADS * EMB), ans_emb, cor_emb)
    return loss.reshape(B), gold.reshape(B), corr.reshape(B)
